# baseline (speedup 1.0000x reference)
import jax
import jax.numpy as jnp
from jax.experimental import pallas as pl
from jax.experimental.pallas import tpu as pltpu

N_ROWS = 1000000
CHUNK = 2500
K_SLOTS = 4
N_CHUNKS = N_ROWS // CHUNK


def _body(lt_any, x_any, out_any, *refs):
    ltb = refs[0:K_SLOTS]
    xb = refs[K_SLOTS : 2 * K_SLOTS]
    ob = refs[2 * K_SLOTS : 3 * K_SLOTS]
    sems = refs[3 * K_SLOTS]

    def in_copies(i):
        s = i % K_SLOTS
        rows = pl.ds(i * CHUNK, CHUNK)
        return (
            pltpu.make_async_copy(lt_any.at[rows], ltb[s], sems.at[0, s]),
            pltpu.make_async_copy(x_any.at[rows], xb[s], sems.at[1, s]),
        )

    def out_copy(i):
        s = i % K_SLOTS
        rows = pl.ds(i * CHUNK, CHUNK)
        return pltpu.make_async_copy(ob[s], out_any.at[rows], sems.at[2, s])

    for k in range(K_SLOTS):
        a, b = in_copies(k)
        a.start()
        b.start()

    for i in range(N_CHUNKS):
        s = i % K_SLOTS
        if i >= K_SLOTS:
            out_copy(i).wait()
        a, b = in_copies(i)
        a.wait()
        b.wait()
        ob[s][:, 0:64] = ltb[s][...]
        ob[s][:, 64:128] = xb[s][...]
        out_copy(i).start()
        if i + K_SLOTS < N_CHUNKS:
            a2, b2 = in_copies(i + K_SLOTS)
            a2.start()
            b2.start()

    for i in range(N_CHUNKS - K_SLOTS, N_CHUNKS):
        out_copy(i).wait()


def kernel(x, adj, lt):
    del adj
    n = lt.shape[0]
    scratch = (
        [pltpu.VMEM((CHUNK, 64), jnp.float32) for _ in range(K_SLOTS)]
        + [pltpu.VMEM((CHUNK, 64), jnp.float32) for _ in range(K_SLOTS)]
        + [pltpu.VMEM((CHUNK, 128), jnp.float32) for _ in range(K_SLOTS)]
        + [pltpu.SemaphoreType.DMA((3, K_SLOTS))]
    )
    return pl.pallas_call(
        _body,
        in_specs=[
            pl.BlockSpec(memory_space=pl.ANY),
            pl.BlockSpec(memory_space=pl.ANY),
        ],
        out_specs=pl.BlockSpec(memory_space=pl.ANY),
        out_shape=jax.ShapeDtypeStruct((n, 128), jnp.float32),
        scratch_shapes=scratch,
    )(lt, x)
